# Initial kernel scaffold; baseline (speedup 1.0000x reference)
import jax
import jax.numpy as jnp
from jax.experimental import pallas as pl


def kernel(positions, embedding):
    raise NotImplementedError("write your pallas kernel here")


# speedup vs baseline: 2.1266x; 2.1266x over previous
import functools

import jax
import jax.numpy as jnp
from jax import lax
from jax.experimental import pallas as pl
from jax.experimental.pallas import tpu as pltpu
from jax.experimental.pallas import tpu_sc as plsc

_CHUNK = 64


@functools.lru_cache(maxsize=None)
def _build(V, D, N, chunk):
    info = plsc.get_sparse_core_info()
    NC, NS = info.num_cores, info.num_subcores
    NW = NC * NS
    rows_per_w = N // NW
    n_chunks = rows_per_w // chunk
    mesh = plsc.VectorSubcoreMesh(core_axis_name="c", subcore_axis_name="s")

    @functools.partial(
        pl.kernel,
        mesh=mesh,
        out_type=jax.ShapeDtypeStruct((N, D), jnp.float32),
        scratch_types=[
            pltpu.VMEM((n_chunks, chunk), jnp.int32),
            pltpu.VMEM((chunk, D), jnp.float32),
            pltpu.SemaphoreType.DMA,
        ],
    )
    def k(idx_hbm, table_hbm, out_hbm, idx_v, rows_v, sem):
        wid = lax.axis_index("s") * NC + lax.axis_index("c")
        base = wid * rows_per_w
        pltpu.sync_copy(idx_hbm.at[pl.ds(wid * n_chunks, n_chunks)], idx_v)
        for j in range(n_chunks):
            pltpu.async_copy(table_hbm.at[idx_v.at[j]], rows_v, sem).wait()
            pltpu.sync_copy(rows_v, out_hbm.at[pl.ds(base + j * chunk, chunk)])

    return k


def kernel(positions, embedding):
    B, T = positions.shape
    V, D = embedding.shape
    N = B * T
    idx = positions.reshape(N // _CHUNK, _CHUNK).astype(jnp.int32)
    out = _build(V, D, N, _CHUNK)(idx, embedding)
    return out.reshape(B, T, D)


# speedup vs baseline: 2.2995x; 1.0813x over previous
import functools

import jax
import jax.numpy as jnp
from jax import lax
from jax.experimental import pallas as pl
from jax.experimental.pallas import tpu as pltpu
from jax.experimental.pallas import tpu_sc as plsc

_CHUNK = 32
_NBUF = 2


@functools.lru_cache(maxsize=None)
def _build(V, D, N, chunk, nbuf):
    info = plsc.get_sparse_core_info()
    NC, NS = info.num_cores, info.num_subcores
    NW = NC * NS
    rows_per_w = N // NW
    n_chunks = rows_per_w // chunk
    mesh = plsc.VectorSubcoreMesh(core_axis_name="c", subcore_axis_name="s")

    @functools.partial(
        pl.kernel,
        mesh=mesh,
        out_type=jax.ShapeDtypeStruct((N, D), jnp.float32),
        scratch_types=[
            pltpu.VMEM((n_chunks, chunk), jnp.int32),
            pltpu.VMEM((nbuf, chunk, D), jnp.float32),
        ]
        + [pltpu.SemaphoreType.DMA] * (2 * nbuf),
    )
    def k(idx_hbm, table_hbm, out_hbm, idx_v, rows_v, *sems):
        gsem, ssem = sems[:nbuf], sems[nbuf:]
        wid = lax.axis_index("s") * NC + lax.axis_index("c")
        base = wid * rows_per_w
        pltpu.sync_copy(idx_hbm.at[pl.ds(wid * n_chunks, n_chunks)], idx_v)
        gets = [None] * nbuf
        puts = [None] * nbuf
        for j in range(min(nbuf, n_chunks)):
            gets[j] = pltpu.async_copy(
                table_hbm.at[idx_v.at[j]], rows_v.at[j], gsem[j])
        for j in range(n_chunks):
            b = j % nbuf
            gets[b].wait()
            puts[b] = pltpu.async_copy(
                rows_v.at[b], out_hbm.at[pl.ds(base + j * chunk, chunk)],
                ssem[b])
            jn = j + nbuf
            if jn < n_chunks:
                puts[b].wait()
                gets[b] = pltpu.async_copy(
                    table_hbm.at[idx_v.at[jn]], rows_v.at[b], gsem[b])
        for j in range(max(0, n_chunks - nbuf), n_chunks):
            puts[j % nbuf].wait()

    return k


def kernel(positions, embedding):
    B, T = positions.shape
    V, D = embedding.shape
    N = B * T
    idx = positions.reshape(N // _CHUNK, _CHUNK).astype(jnp.int32)
    out = _build(V, D, N, _CHUNK, _NBUF)(idx, embedding)
    return out.reshape(B, T, D)


# speedup vs baseline: 2.3256x; 1.0113x over previous
import functools

import jax
import jax.numpy as jnp
from jax import lax
from jax.experimental import pallas as pl
from jax.experimental.pallas import tpu as pltpu
from jax.experimental.pallas import tpu_sc as plsc

_CHUNK = 32
_NBUF = 3


@functools.lru_cache(maxsize=None)
def _build(V, D, N, chunk, nbuf):
    info = plsc.get_sparse_core_info()
    NC, NS = info.num_cores, info.num_subcores
    NW = NC * NS
    rows_per_w = N // NW
    n_chunks = rows_per_w // chunk
    mesh = plsc.VectorSubcoreMesh(core_axis_name="c", subcore_axis_name="s")

    @functools.partial(
        pl.kernel,
        mesh=mesh,
        out_type=jax.ShapeDtypeStruct((N, D), jnp.float32),
        scratch_types=[
            pltpu.VMEM((n_chunks, chunk), jnp.int32),
            pltpu.VMEM((nbuf, chunk, D), jnp.float32),
        ]
        + [pltpu.SemaphoreType.DMA] * (2 * nbuf),
    )
    def k(idx_hbm, table_hbm, out_hbm, idx_v, rows_v, *sems):
        gsem, ssem = sems[:nbuf], sems[nbuf:]
        wid = lax.axis_index("s") * NC + lax.axis_index("c")
        base = wid * rows_per_w
        pltpu.sync_copy(idx_hbm.at[pl.ds(wid * n_chunks, n_chunks)], idx_v)
        gets = [None] * nbuf
        puts = [None] * nbuf
        for j in range(min(nbuf, n_chunks)):
            gets[j] = pltpu.async_copy(
                table_hbm.at[idx_v.at[j]], rows_v.at[j], gsem[j])
        for j in range(n_chunks):
            b = j % nbuf
            gets[b].wait()
            puts[b] = pltpu.async_copy(
                rows_v.at[b], out_hbm.at[pl.ds(base + j * chunk, chunk)],
                ssem[b])
            jn = j + nbuf
            if jn < n_chunks:
                puts[b].wait()
                gets[b] = pltpu.async_copy(
                    table_hbm.at[idx_v.at[jn]], rows_v.at[b], gsem[b])
        for j in range(max(0, n_chunks - nbuf), n_chunks):
            puts[j % nbuf].wait()

    return k


def kernel(positions, embedding):
    B, T = positions.shape
    V, D = embedding.shape
    N = B * T
    idx = positions.reshape(N // _CHUNK, _CHUNK).astype(jnp.int32)
    out = _build(V, D, N, _CHUNK, _NBUF)(idx, embedding)
    return out.reshape(B, T, D)
